# Initial kernel scaffold; baseline (speedup 1.0000x reference)
import jax
import jax.numpy as jnp
from jax.experimental import pallas as pl


def kernel(input, embedding, fusion_embedding):
    raise NotImplementedError("write your pallas kernel here")


# speedup vs baseline: 8.4425x; 8.4425x over previous
import functools

import jax
import jax.numpy as jnp
from jax import lax
from jax.experimental import pallas as pl
from jax.experimental.pallas import tpu as pltpu
from jax.experimental.pallas import tpu_sc as plsc

VOCAB = 100000
FUSION_VOCAB = 256
DIM = 128
B = 1024
S = 200
N = B * S

NC = 2
NS = 16
NW = NC * NS
CHUNK = 256
N_PER_W = N // NW
N_CHUNKS = N_PER_W // CHUNK


def _sc_body(tok_hbm, emb_hbm, fus_hbm, out_hbm, tok_v, midx_v, stage_v,
             fus_v, sem):
    wid = lax.axis_index("s") * NC + lax.axis_index("c")
    base = wid * N_PER_W

    pltpu.sync_copy(fus_hbm, fus_v)

    def chunk_body(k, carry):
        off = base + k * CHUNK
        pltpu.sync_copy(tok_hbm.at[pl.ds(off, CHUNK)], tok_v)

        for g in range(CHUNK // 16):
            t = tok_v[pl.ds(g * 16, 16)]
            midx_v[g // 8, pl.ds((g % 8) * 16, 16)] = jnp.where(t < VOCAB, t, 0)

        cp0 = pltpu.async_copy(emb_hbm.at[midx_v.at[0]],
                               stage_v.at[pl.ds(0, 128)], sem)
        cp1 = pltpu.async_copy(emb_hbm.at[midx_v.at[1]],
                               stage_v.at[pl.ds(128, 128)], sem)
        cp0.wait()
        cp1.wait()

        def fix_group(g, c2):
            t = tok_v[pl.ds(g * 16, 16)]
            fmask = t >= VOCAB
            anyf = plsc.all_reduce_population_count(fmask)[0]

            @pl.when(anyf > 0)
            def _():
                fidx = jnp.where(fmask, t - VOCAB, 0)
                rows = g * 16 + lax.iota(jnp.int32, 16)

                def fix_col(c, c3):
                    cols = jnp.full((16,), c, jnp.int32)
                    vals = plsc.load_gather(fus_v, [fidx, cols])
                    plsc.store_scatter(stage_v, [rows, cols], vals,
                                       mask=fmask)
                    return c3

                lax.fori_loop(0, DIM, fix_col, 0)
            return c2

        lax.fori_loop(0, CHUNK // 16, fix_group, 0)

        pltpu.sync_copy(stage_v, out_hbm.at[pl.ds(off, CHUNK)])
        return carry

    lax.fori_loop(0, N_CHUNKS, chunk_body, 0)


@functools.partial(
    pl.kernel,
    mesh=plsc.VectorSubcoreMesh(core_axis_name="c", subcore_axis_name="s"),
    out_type=jax.ShapeDtypeStruct((N, DIM), jnp.float32),
    compiler_params=pltpu.CompilerParams(needs_layout_passes=False),
    scratch_types=[
        pltpu.VMEM((CHUNK,), jnp.int32),
        pltpu.VMEM((2, 128), jnp.int32),
        pltpu.VMEM((CHUNK, DIM), jnp.float32),
        pltpu.VMEM((FUSION_VOCAB, DIM), jnp.float32),
        pltpu.SemaphoreType.DMA,
    ],
)
def _sc_embed(tok_hbm, emb_hbm, fus_hbm, out_hbm, tok_v, midx_v, stage_v,
              fus_v, sem):
    _sc_body(tok_hbm, emb_hbm, fus_hbm, out_hbm, tok_v, midx_v, stage_v,
             fus_v, sem)


def kernel(input, embedding, fusion_embedding):
    flat = input.reshape(N)
    out = _sc_embed(flat, embedding, fusion_embedding)
    return out.reshape(B, S, DIM)
